# baseline (speedup 1.0000x reference)
import functools
import math

import jax
import jax.numpy as jnp
from jax.experimental import pallas as pl
from jax.experimental.pallas import tpu as pltpu

_IN_CH = 32
_OUT_CH = 32
_K = 8
_SCALE = 4
_H = 128
_W = 128
_C1 = (_IN_CH + 2) * _SCALE * _SCALE
_CX = _IN_CH * _SCALE * _SCALE
_CC = 2 * _SCALE * _SCALE
_N = (_H * _W) // (_SCALE * _SCALE)
_P = _SCALE * _SCALE
_CF = _OUT_CH * _P


def _fold_kernel(cw2_ref, cb2_ref, pw_w_ref, pw_b_ref, wfin_ref, bfin_ref):
    pw = pw_w_ref[...]
    ii = jax.lax.broadcasted_iota(jnp.int32, (_C1, _CF), 0)
    qq = jax.lax.broadcasted_iota(jnp.int32, (_C1, _CF), 1)
    same_p = (ii % _P) == (qq % _P)
    oc_c = (jax.lax.broadcasted_iota(jnp.int32, (_C1, _IN_CH + 2), 0) // _P ==
            jax.lax.broadcasted_iota(jnp.int32, (_C1, _IN_CH + 2), 1)
            ).astype(jnp.float32)
    oc_o = (jax.lax.broadcasted_iota(jnp.int32, (_OUT_CH, _CF), 1) // _P ==
            jax.lax.broadcasted_iota(jnp.int32, (_OUT_CH, _CF), 0)
            ).astype(jnp.float32)
    pw_big = jnp.dot(jnp.dot(oc_c, pw.T, preferred_element_type=jnp.float32),
                     oc_o, preferred_element_type=jnp.float32)
    pw_big = jnp.where(same_p, pw_big, 0.0)
    wfin_ref[...] = jax.lax.dot_general(
        cw2_ref[...], pw_big, (((0,), (0,)), ((), ())),
        preferred_element_type=jnp.float32)
    bf2 = jnp.dot(pw, cb2_ref[...], preferred_element_type=jnp.float32)
    bfin_ref[...] = bf2 + pw_b_ref[...].reshape(_OUT_CH, 1)


def _coords_tokens():
    r = jax.lax.broadcasted_iota(jnp.int32, (_N, _CC), 0)
    q = jax.lax.broadcasted_iota(jnp.int32, (_N, _CC), 1)
    d = q // _P
    sh = (q % _P) // _SCALE
    sw = q % _SCALE
    i = (_SCALE * (r // (_W // _SCALE)) + sh).astype(jnp.float32)
    j = (_SCALE * (r % (_W // _SCALE)) + sw).astype(jnp.float32)
    norm = jnp.maximum(jnp.sqrt(i * i + j * j), 1e-12)
    return jnp.where(d == 0, i, j) / norm


def _attn_kernel(x_ref, wq_ref, bq_ref, wk_ref, bk_ref, wv_ref, bv_ref,
                 wfin_ref, bfin_ref, out_ref):
    x = x_ref[0]
    ct = _coords_tokens()

    def proj(w_ref, b_ref):
        wx = w_ref[:, :_CX]
        wc = w_ref[:, _CX:]
        y = jax.lax.dot_general(x, wx, (((1,), (1,)), ((), ())),
                                preferred_element_type=jnp.float32)
        y = y + jax.lax.dot_general(ct, wc, (((1,), (1,)), ((), ())),
                                    preferred_element_type=jnp.float32)
        return y + b_ref[...]

    q = proj(wq_ref, bq_ref) * (1.0 / math.sqrt(_C1))
    k = proj(wk_ref, bk_ref)
    v = proj(wv_ref, bv_ref)
    sim = jax.lax.dot_general(q, k, (((1,), (1,)), ((), ())),
                              preferred_element_type=jnp.float32)

    iota_m = jax.lax.broadcasted_iota(jnp.int32, (_N, _N), 1)
    topv = []
    topi = []
    work = sim
    for _ in range(_K):
        mx = jnp.max(work, axis=1, keepdims=True)
        idx = jnp.min(jnp.where(work == mx, iota_m, _N), axis=1,
                      keepdims=True)
        topv.append(mx)
        topi.append(idx)
        work = jnp.where(iota_m == idx, -jnp.inf, work)

    exps = [jnp.exp(tv - topv[0]) for tv in topv]
    denom = functools.reduce(lambda a, b: a + b, exps)
    inv = 1.0 / denom

    u_all = jax.lax.dot_general(
        v.astype(jnp.bfloat16), wfin_ref[...].astype(jnp.bfloat16),
        (((1,), (0,)), ((), ())), preferred_element_type=jnp.float32)

    acc = jnp.broadcast_to(bfin_ref[...], (_N, _CF))
    for kk in range(_K):
        u_k = u_all[:, kk * _CF:(kk + 1) * _CF]
        a_k = exps[kk] * inv
        p_k = jnp.where(iota_m == topi[kk], 1.0, 0.0).astype(jnp.bfloat16)
        g_k = jnp.dot(p_k, u_k.astype(jnp.bfloat16),
                      preferred_element_type=jnp.float32)
        acc = acc + g_k * a_k
    out_ref[0] = acc


def kernel(x, Wq, bq, Wk, bk, Wv, bv, conv_w, conv_b, pw_w, pw_b):
    b = x.shape[0]
    x1 = x.reshape(b, _IN_CH, _H // _SCALE, _SCALE, _W // _SCALE, _SCALE)
    x2t = x1.transpose(0, 2, 4, 1, 3, 5).reshape(b, _N, _CX)

    cw2 = conv_w.reshape(_C1, _C1 * _K)
    cb2 = conv_b.reshape(_IN_CH + 2, _P)
    wfin_all, bfin2 = pl.pallas_call(
        _fold_kernel,
        out_shape=(
            jax.ShapeDtypeStruct((_C1 * _K, _CF), jnp.float32),
            jax.ShapeDtypeStruct((_OUT_CH, _P), jnp.float32),
        ),
    )(cw2, cb2, pw_w, pw_b.reshape(_OUT_CH, 1))
    wfin2 = wfin_all.reshape(_C1, _K * _CF)
    bfin = bfin2.reshape(1, _CF)

    final = pl.pallas_call(
        _attn_kernel,
        grid=(b,),
        in_specs=[
            pl.BlockSpec((1, _N, _CX), lambda i: (i, 0, 0)),
            pl.BlockSpec((_C1, _C1), lambda i: (0, 0)),
            pl.BlockSpec((1, _C1), lambda i: (0, 0)),
            pl.BlockSpec((_C1, _C1), lambda i: (0, 0)),
            pl.BlockSpec((1, _C1), lambda i: (0, 0)),
            pl.BlockSpec((_C1, _C1), lambda i: (0, 0)),
            pl.BlockSpec((1, _C1), lambda i: (0, 0)),
            pl.BlockSpec((_C1, _K * _CF), lambda i: (0, 0)),
            pl.BlockSpec((1, _CF), lambda i: (0, 0)),
        ],
        out_specs=pl.BlockSpec((1, _N, _CF), lambda i: (i, 0, 0)),
        out_shape=jax.ShapeDtypeStruct((b, _N, _CF), jnp.float32),
        compiler_params=pltpu.CompilerParams(
            dimension_semantics=("parallel",)),
    )(x2t, Wq, bq.reshape(1, _C1), Wk, bk.reshape(1, _C1),
      Wv, bv.reshape(1, _C1), wfin2, bfin)

    out = final.reshape(b, _H // _SCALE, _W // _SCALE, _OUT_CH, _SCALE, _SCALE)
    out = out.transpose(0, 3, 1, 4, 2, 5).reshape(b, _OUT_CH, _H, _W)
    return out


# speedup vs baseline: 1.0265x; 1.0265x over previous
import functools
import math

import jax
import jax.numpy as jnp
from jax.experimental import pallas as pl
from jax.experimental.pallas import tpu as pltpu

_IN_CH = 32
_OUT_CH = 32
_K = 8
_SCALE = 4
_H = 128
_W = 128
_C1 = (_IN_CH + 2) * _SCALE * _SCALE
_CX = _IN_CH * _SCALE * _SCALE
_CC = 2 * _SCALE * _SCALE
_N = (_H * _W) // (_SCALE * _SCALE)
_P = _SCALE * _SCALE
_CF = _OUT_CH * _P


def _fold_kernel(cw2_ref, cb2_ref, pw_w_ref, pw_b_ref, wfin_ref, bfin_ref):
    pw = pw_w_ref[...]
    ii = jax.lax.broadcasted_iota(jnp.int32, (_C1, _CF), 0)
    qq = jax.lax.broadcasted_iota(jnp.int32, (_C1, _CF), 1)
    same_p = (ii % _P) == (qq % _P)
    oc_c = (jax.lax.broadcasted_iota(jnp.int32, (_C1, _IN_CH + 2), 0) // _P ==
            jax.lax.broadcasted_iota(jnp.int32, (_C1, _IN_CH + 2), 1)
            ).astype(jnp.float32)
    oc_o = (jax.lax.broadcasted_iota(jnp.int32, (_OUT_CH, _CF), 1) // _P ==
            jax.lax.broadcasted_iota(jnp.int32, (_OUT_CH, _CF), 0)
            ).astype(jnp.float32)
    hi = jax.lax.Precision.HIGHEST
    pw_big = jnp.dot(jnp.dot(oc_c, pw.T, precision=hi,
                             preferred_element_type=jnp.float32),
                     oc_o, precision=hi,
                     preferred_element_type=jnp.float32)
    pw_big = jnp.where(same_p, pw_big, 0.0)
    for k in range(_K):
        wfin_ref[k] = jnp.dot(cw2_ref[k], pw_big, precision=hi,
                              preferred_element_type=jnp.float32)
    bf2 = jnp.dot(pw, cb2_ref[...], precision=hi,
                  preferred_element_type=jnp.float32)
    bfin_ref[...] = bf2 + pw_b_ref[...].reshape(_OUT_CH, 1)


def _coords_tokens():
    r = jax.lax.broadcasted_iota(jnp.int32, (_N, _CC), 0)
    q = jax.lax.broadcasted_iota(jnp.int32, (_N, _CC), 1)
    d = q // _P
    sh = (q % _P) // _SCALE
    sw = q % _SCALE
    i = (_SCALE * (r // (_W // _SCALE)) + sh).astype(jnp.float32)
    j = (_SCALE * (r % (_W // _SCALE)) + sw).astype(jnp.float32)
    norm = jnp.maximum(jnp.sqrt(i * i + j * j), 1e-12)
    return jnp.where(d == 0, i, j) / norm


def _attn_kernel(x_ref, wq_ref, bq_ref, wk_ref, bk_ref, wv_ref, bv_ref,
                 wfin_ref, bfin_ref, out_ref):
    xb = x_ref[0].astype(jnp.bfloat16)

    def proj(w_ref, b_ref):
        y = jax.lax.dot_general(xb, w_ref[...].astype(jnp.bfloat16),
                                (((1,), (1,)), ((), ())),
                                preferred_element_type=jnp.float32)
        return y + b_ref[...]

    q = proj(wq_ref, bq_ref)
    k = proj(wk_ref, bk_ref)
    v = proj(wv_ref, bv_ref)
    sim = jax.lax.dot_general(q.astype(jnp.bfloat16), k.astype(jnp.bfloat16),
                              (((1,), (1,)), ((), ())),
                              preferred_element_type=jnp.float32)

    iota_m = jax.lax.broadcasted_iota(jnp.int32, (_N, _N), 1)
    topv = []
    topi = []
    work = sim
    for _ in range(_K):
        mx = jnp.max(work, axis=1, keepdims=True)
        idx = jnp.min(jnp.where(work == mx, iota_m, _N), axis=1,
                      keepdims=True)
        topv.append(mx)
        topi.append(idx)
        work = jnp.where(iota_m == idx, -jnp.inf, work)

    exps = [jnp.exp((tv - topv[0]) * (1.0 / math.sqrt(_C1))) for tv in topv]
    denom = functools.reduce(lambda a, b: a + b, exps)
    inv = 1.0 / denom

    acc = jnp.broadcast_to(bfin_ref[...], (_N, _CF))
    vb = v.astype(jnp.bfloat16)
    for kk in range(_K):
        u_k = jnp.dot(vb, wfin_ref[kk].astype(jnp.bfloat16),
                      preferred_element_type=jnp.float32)
        a_k = exps[kk] * inv
        p_k = jnp.where(iota_m == topi[kk], 1.0, 0.0).astype(jnp.bfloat16)
        g_k = jnp.dot(p_k, u_k.astype(jnp.bfloat16),
                      preferred_element_type=jnp.float32)
        acc = acc + g_k * a_k
    out_ref[0] = acc


def kernel(x, Wq, bq, Wk, bk, Wv, bv, conv_w, conv_b, pw_w, pw_b):
    b = x.shape[0]
    x1 = x.reshape(b, _IN_CH, _H // _SCALE, _SCALE, _W // _SCALE, _SCALE)
    x2t = x1.transpose(0, 2, 4, 1, 3, 5).reshape(b, _N, _CX)
    ct = jnp.broadcast_to(_coords_tokens()[None], (b, _N, _CC))
    x2t = jnp.concatenate([x2t, ct], axis=2)

    cw2 = conv_w.transpose(2, 1, 0)
    cb2 = conv_b.reshape(_IN_CH + 2, _P)
    wfin2, bfin2 = pl.pallas_call(
        _fold_kernel,
        out_shape=(
            jax.ShapeDtypeStruct((_K, _C1, _CF), jnp.float32),
            jax.ShapeDtypeStruct((_OUT_CH, _P), jnp.float32),
        ),
    )(cw2, cb2, pw_w, pw_b.reshape(_OUT_CH, 1))
    bfin = bfin2.reshape(1, _CF)

    final = pl.pallas_call(
        _attn_kernel,
        grid=(b,),
        in_specs=[
            pl.BlockSpec((1, _N, _C1), lambda i: (i, 0, 0)),
            pl.BlockSpec((_C1, _C1), lambda i: (0, 0)),
            pl.BlockSpec((1, _C1), lambda i: (0, 0)),
            pl.BlockSpec((_C1, _C1), lambda i: (0, 0)),
            pl.BlockSpec((1, _C1), lambda i: (0, 0)),
            pl.BlockSpec((_C1, _C1), lambda i: (0, 0)),
            pl.BlockSpec((1, _C1), lambda i: (0, 0)),
            pl.BlockSpec((_K, _C1, _CF), lambda i: (0, 0, 0)),
            pl.BlockSpec((1, _CF), lambda i: (0, 0)),
        ],
        out_specs=pl.BlockSpec((1, _N, _CF), lambda i: (i, 0, 0)),
        out_shape=jax.ShapeDtypeStruct((b, _N, _CF), jnp.float32),
    )(x2t, Wq, bq.reshape(1, _C1), Wk, bk.reshape(1, _C1),
      Wv, bv.reshape(1, _C1), wfin2, bfin)

    out = final.reshape(b, _H // _SCALE, _W // _SCALE, _OUT_CH, _SCALE, _SCALE)
    out = out.transpose(0, 3, 1, 4, 2, 5).reshape(b, _OUT_CH, _H, _W)
    return out


# speedup vs baseline: 1.1078x; 1.0792x over previous
import functools
import math

import jax
import jax.numpy as jnp
from jax.experimental import pallas as pl
from jax.experimental.pallas import tpu as pltpu

_IN_CH = 32
_OUT_CH = 32
_K = 8
_SCALE = 4
_H = 128
_W = 128
_C1 = (_IN_CH + 2) * _SCALE * _SCALE
_CX = _IN_CH * _SCALE * _SCALE
_CC = 2 * _SCALE * _SCALE
_N = (_H * _W) // (_SCALE * _SCALE)
_P = _SCALE * _SCALE
_CF = _OUT_CH * _P


def _fold_kernel(cw2_ref, cb2_ref, pw_w_ref, pw_b_ref, wfin_ref, bfin_ref):
    pw = pw_w_ref[...]
    ii = jax.lax.broadcasted_iota(jnp.int32, (_C1, _CF), 0)
    qq = jax.lax.broadcasted_iota(jnp.int32, (_C1, _CF), 1)
    same_p = (ii % _P) == (qq % _P)
    oc_c = (jax.lax.broadcasted_iota(jnp.int32, (_C1, _IN_CH + 2), 0) // _P ==
            jax.lax.broadcasted_iota(jnp.int32, (_C1, _IN_CH + 2), 1)
            ).astype(jnp.float32)
    oc_o = (jax.lax.broadcasted_iota(jnp.int32, (_OUT_CH, _CF), 1) // _P ==
            jax.lax.broadcasted_iota(jnp.int32, (_OUT_CH, _CF), 0)
            ).astype(jnp.float32)
    pw_big = jnp.dot(jnp.dot(oc_c, pw.T, preferred_element_type=jnp.float32),
                     oc_o, preferred_element_type=jnp.float32)
    pw_big = jnp.where(same_p, pw_big, 0.0)
    for k in range(_K):
        wfin_ref[k] = jnp.dot(cw2_ref[k], pw_big,
                              preferred_element_type=jnp.float32)
    bf2 = jnp.dot(pw, cb2_ref[...], preferred_element_type=jnp.float32)
    bfin_ref[...] = bf2 + pw_b_ref[...].reshape(_OUT_CH, 1)


def _coords_tokens():
    r = jax.lax.broadcasted_iota(jnp.int32, (_N, _CC), 0)
    q = jax.lax.broadcasted_iota(jnp.int32, (_N, _CC), 1)
    d = q // _P
    sh = (q % _P) // _SCALE
    sw = q % _SCALE
    i = (_SCALE * (r // (_W // _SCALE)) + sh).astype(jnp.float32)
    j = (_SCALE * (r % (_W // _SCALE)) + sw).astype(jnp.float32)
    norm = jnp.maximum(jnp.sqrt(i * i + j * j), 1e-12)
    return jnp.where(d == 0, i, j) / norm


def _attn_kernel(x_ref, wq_ref, bq_ref, wk_ref, bk_ref, wv_ref, bv_ref,
                 wfin_ref, bfin_ref, out_ref):
    xb = x_ref[0]

    def proj(w_ref, b_ref):
        y = jax.lax.dot_general(xb, w_ref[...].astype(jnp.bfloat16),
                                (((1,), (1,)), ((), ())),
                                preferred_element_type=jnp.float32)
        return y + b_ref[...]

    q = proj(wq_ref, bq_ref)
    k = proj(wk_ref, bk_ref)
    v = proj(wv_ref, bv_ref)
    sim = jax.lax.dot_general(q.astype(jnp.bfloat16), k.astype(jnp.bfloat16),
                              (((1,), (1,)), ((), ())),
                              preferred_element_type=jnp.float32)

    iota_m = jax.lax.broadcasted_iota(jnp.int32, (_N, _N), 1)
    topv = []
    topi = []
    work = sim
    for _ in range(_K):
        mx = jnp.max(work, axis=1, keepdims=True)
        idx = jnp.min(jnp.where(work == mx, iota_m, _N), axis=1,
                      keepdims=True)
        topv.append(mx)
        topi.append(idx)
        work = jnp.where(iota_m == idx, -jnp.inf, work)

    exps = [jnp.exp((tv - topv[0]) * (1.0 / math.sqrt(_C1))) for tv in topv]
    denom = functools.reduce(lambda a, b: a + b, exps)
    inv = 1.0 / denom

    acc = jnp.broadcast_to(bfin_ref[...], (_N, _CF))
    vb = v.astype(jnp.bfloat16)
    for kk in range(_K):
        u_k = jnp.dot(vb, wfin_ref[kk].astype(jnp.bfloat16),
                      preferred_element_type=jnp.float32)
        a_k = exps[kk] * inv
        p_k = jnp.where(iota_m == topi[kk], 1.0, 0.0).astype(jnp.bfloat16)
        g_k = jnp.dot(p_k, u_k.astype(jnp.bfloat16),
                      preferred_element_type=jnp.float32)
        acc = acc + g_k * a_k
    out_ref[0] = acc


def kernel(x, Wq, bq, Wk, bk, Wv, bv, conv_w, conv_b, pw_w, pw_b):
    b = x.shape[0]
    xh = x.astype(jnp.bfloat16)
    x1 = xh.reshape(b, _IN_CH, _H // _SCALE, _SCALE, _W // _SCALE, _SCALE)
    x2t = x1.transpose(0, 2, 4, 1, 3, 5).reshape(b, _N, _CX)
    ct = jnp.broadcast_to(_coords_tokens()[None].astype(jnp.bfloat16),
                          (b, _N, _CC))
    x2t = jnp.concatenate([x2t, ct], axis=2)

    cw2 = conv_w.transpose(2, 1, 0)
    cb2 = conv_b.reshape(_IN_CH + 2, _P)
    wfin2, bfin2 = pl.pallas_call(
        _fold_kernel,
        out_shape=(
            jax.ShapeDtypeStruct((_K, _C1, _CF), jnp.float32),
            jax.ShapeDtypeStruct((_OUT_CH, _P), jnp.float32),
        ),
    )(cw2, cb2, pw_w, pw_b.reshape(_OUT_CH, 1))
    bfin = bfin2.reshape(1, _CF)

    final = pl.pallas_call(
        _attn_kernel,
        grid=(b,),
        in_specs=[
            pl.BlockSpec((1, _N, _C1), lambda i: (i, 0, 0)),
            pl.BlockSpec((_C1, _C1), lambda i: (0, 0)),
            pl.BlockSpec((1, _C1), lambda i: (0, 0)),
            pl.BlockSpec((_C1, _C1), lambda i: (0, 0)),
            pl.BlockSpec((1, _C1), lambda i: (0, 0)),
            pl.BlockSpec((_C1, _C1), lambda i: (0, 0)),
            pl.BlockSpec((1, _C1), lambda i: (0, 0)),
            pl.BlockSpec((_K, _C1, _CF), lambda i: (0, 0, 0)),
            pl.BlockSpec((1, _CF), lambda i: (0, 0)),
        ],
        out_specs=pl.BlockSpec((1, _N, _CF), lambda i: (i, 0, 0)),
        out_shape=jax.ShapeDtypeStruct((b, _N, _CF), jnp.float32),
    )(x2t, Wq, bq.reshape(1, _C1), Wk, bk.reshape(1, _C1),
      Wv, bv.reshape(1, _C1), wfin2, bfin)

    out = final.reshape(b, _H // _SCALE, _W // _SCALE, _OUT_CH, _SCALE, _SCALE)
    out = out.transpose(0, 3, 1, 4, 2, 5).reshape(b, _OUT_CH, _H, _W)
    return out
